# baseline (speedup 1.0000x reference)
import functools

import jax
import jax.numpy as jnp
from jax import lax
from jax.experimental import pallas as pl
from jax.experimental.pallas import tpu as pltpu
from jax.experimental.pallas import tpu_sc as plsc

VOCAB = 1000
EMBED = 4
BATCH = 1024
HIST = 50
NTOK = BATCH * HIST

NC, NS = 2, 16
NW = NC * NS
B_PER_W = NTOK // NW
CHUNK = 40
N_CHUNKS = B_PER_W // CHUNK


def _proj_body(table_ref, w_ref, b_ref, p_ref):
    p = lax.dot_general(
        table_ref[...], w_ref[...],
        dimension_numbers=(((1,), (1,)), ((), ())),
        preferred_element_type=jnp.float32,
    )
    p_ref[...] = p + b_ref[...]


def _compute_p(table, W, b):
    return pl.pallas_call(
        _proj_body,
        out_shape=jax.ShapeDtypeStruct((VOCAB, VOCAB), jnp.float32),
    )(table, W, b.reshape(1, VOCAB))


def _gather_body(p_hbm, x_hbm, out_hbm, idx_v, rows0, rows1,
                 sg0, sg1, ss0, ss1):
    wid = lax.axis_index("s") * NC + lax.axis_index("c")
    base = wid * B_PER_W
    pltpu.sync_copy(x_hbm.at[pl.ds(base, B_PER_W)], idx_v)
    rows, sg, ss = (rows0, rows1), (sg0, sg1), (ss0, ss1)

    def g_desc(g, b):
        idx_c = idx_v.at[pl.ds(g * CHUNK, CHUNK)]
        return pltpu.make_async_copy(p_hbm.at[idx_c], rows[b], sg[b])

    def s_desc(g, b):
        dst = out_hbm.at[pl.ds(base + g * CHUNK, CHUNK)]
        return pltpu.make_async_copy(rows[b], dst, ss[b])

    g_desc(0, 0).start()
    g_desc(1, 1).start()
    ni = N_CHUNKS // 2

    def body(i, _):
        g = 2 * i
        for b in range(2):
            g_desc(g + b, b).wait()
            s_desc(g + b, b).start()

        @pl.when(i < ni - 1)
        def _refill():
            for b in range(2):
                s_desc(g + b, b).wait()
                g_desc(g + 2 + b, b).start()

        @pl.when(i == ni - 1)
        def _drain():
            for b in range(2):
                s_desc(g + b, b).wait()

        return 0

    lax.fori_loop(0, ni, body, 0)


_gather = functools.partial(
    pl.kernel,
    mesh=plsc.VectorSubcoreMesh(core_axis_name="c", subcore_axis_name="s"),
    out_type=jax.ShapeDtypeStruct((NTOK, VOCAB), jnp.float32),
    compiler_params=pltpu.CompilerParams(use_tc_tiling_on_sc=False),
    scratch_types=[
        pltpu.VMEM((B_PER_W,), jnp.int32),
        pltpu.VMEM((CHUNK, VOCAB), jnp.float32),
        pltpu.VMEM((CHUNK, VOCAB), jnp.float32),
        pltpu.SemaphoreType.DMA,
        pltpu.SemaphoreType.DMA,
        pltpu.SemaphoreType.DMA,
        pltpu.SemaphoreType.DMA,
    ],
)(_gather_body)


def kernel(x, table, W, b):
    p = _compute_p(table, W, b)
    x_flat = x.reshape(NTOK).astype(jnp.int32)
    out = _gather(p, x_flat)
    return out.reshape(BATCH, HIST, VOCAB)


# speedup vs baseline: 1.7482x; 1.7482x over previous
import functools

import jax
import jax.numpy as jnp
from jax import lax
from jax.experimental import pallas as pl
from jax.experimental.pallas import tpu as pltpu
from jax.experimental.pallas import tpu_sc as plsc

VOCAB = 1000
VPAD = 1024
EMBED = 4
BATCH = 1024
HIST = 50

NC, NS = 2, 16
NW = NC * NS
NB_PER_W = BATCH // NW


def _proj_body(table_ref, w_ref, b_ref, p_ref):
    p = lax.dot_general(
        table_ref[...], w_ref[...],
        dimension_numbers=(((1,), (1,)), ((), ())),
        preferred_element_type=jnp.float32,
    )
    p_ref[...] = p + b_ref[...]


def _compute_p(table, W, b):
    wp = jnp.pad(W, ((0, VPAD - VOCAB), (0, 0)))
    bp = jnp.pad(b, (0, VPAD - VOCAB))
    return pl.pallas_call(
        _proj_body,
        out_shape=jax.ShapeDtypeStruct((VOCAB, VPAD), jnp.float32),
    )(table, wp, bp.reshape(1, VPAD))


MAIN = 896
TAIL = VOCAB - MAIN


def _gather_body(p_hbm, x_hbm, out_hbm, idx_v, rows0, rows1, rows2_0, rows2_1,
                 tail_v, sg0, sg1, ss0, ss1, st):
    wid = lax.axis_index("s") * NC + lax.axis_index("c")
    wb = wid * NB_PER_W
    pltpu.sync_copy(x_hbm.at[pl.ds(wb, NB_PER_W)], idx_v)
    rows, rows2 = (rows0, rows1), (rows2_0, rows2_1)
    sg, ss = (sg0, sg1), (ss0, ss1)

    def g_desc(i, b):
        idx48 = idx_v.at[i].at[pl.ds(0, 48)]
        return pltpu.make_async_copy(p_hbm.at[idx48], rows[b], sg[b])

    def g2_desc(i, b):
        idx2 = idx_v.at[i].at[pl.ds(48, 2)]
        return pltpu.make_async_copy(p_hbm.at[idx2], rows2[b], sg[b])

    def s_desc(i, b):
        src = rows[b].at[:, pl.ds(0, MAIN)]
        dst = out_hbm.at[wb + i].at[pl.ds(0, 48)].at[:, pl.ds(0, MAIN)]
        return pltpu.make_async_copy(src, dst, ss[b])

    def s2_desc(i, b):
        src = rows2[b].at[:, pl.ds(0, MAIN)]
        dst = out_hbm.at[wb + i].at[pl.ds(48, 2)].at[:, pl.ds(0, MAIN)]
        return pltpu.make_async_copy(src, dst, ss[b])

    def t_desc(i):
        dst = out_hbm.at[wb + i].at[:, pl.ds(MAIN, TAIL)]
        return pltpu.make_async_copy(tail_v, dst, st)

    def copy_tail(b):
        for l in range(48):
            for k in range(7):
                s = min(16 * k, TAIL - 16)
                tail_v[l, pl.ds(s, 16)] = rows[b][l, pl.ds(MAIN + s, 16)]
        for l in range(2):
            for k in range(7):
                s = min(16 * k, TAIL - 16)
                tail_v[48 + l, pl.ds(s, 16)] = rows2[b][l, pl.ds(MAIN + s, 16)]

    for b in range(2):
        g_desc(b, b).start()
        g2_desc(b, b).start()
    ni = NB_PER_W // 2

    def body(i, _):
        g = 2 * i
        for b in range(2):
            g_desc(g + b, b).wait()
            g2_desc(g + b, b).wait()
            s_desc(g + b, b).start()
            s2_desc(g + b, b).start()
            copy_tail(b)
            t_desc(g + b).start()
            t_desc(g + b).wait()

        @pl.when(i < ni - 1)
        def _refill():
            for b in range(2):
                s_desc(g + b, b).wait()
                s2_desc(g + b, b).wait()
                g_desc(g + 2 + b, b).start()
                g2_desc(g + 2 + b, b).start()

        @pl.when(i == ni - 1)
        def _drain():
            for b in range(2):
                s_desc(g + b, b).wait()
                s2_desc(g + b, b).wait()

        return 0

    lax.fori_loop(0, ni, body, 0)


_gather = functools.partial(
    pl.kernel,
    mesh=plsc.VectorSubcoreMesh(core_axis_name="c", subcore_axis_name="s"),
    out_type=jax.ShapeDtypeStruct((BATCH, HIST, VOCAB), jnp.float32),
    scratch_types=[
        pltpu.VMEM((NB_PER_W, HIST), jnp.int32),
        pltpu.VMEM((48, VPAD), jnp.float32),
        pltpu.VMEM((48, VPAD), jnp.float32),
        pltpu.VMEM((2, VPAD), jnp.float32),
        pltpu.VMEM((2, VPAD), jnp.float32),
        pltpu.VMEM((HIST, TAIL), jnp.float32),
        pltpu.SemaphoreType.DMA,
        pltpu.SemaphoreType.DMA,
        pltpu.SemaphoreType.DMA,
        pltpu.SemaphoreType.DMA,
        pltpu.SemaphoreType.DMA,
    ],
)(_gather_body)


def kernel(x, table, W, b):
    p = _compute_p(table, W, b)
    return _gather(p, x.astype(jnp.int32))
